# baseline (speedup 1.0000x reference)
import functools

import jax
import jax.numpy as jnp
import numpy as np
from jax.experimental import pallas as pl
from jax.sharding import Mesh, PartitionSpec as P

try:
    from jax.experimental.shard_map import shard_map
except ImportError:
    from jax import shard_map


def _xw_kernel(x_ref, w_ref, o_ref):
    o_ref[...] = jnp.dot(x_ref[...], w_ref[...],
                         preferred_element_type=jnp.float32)


def _pass1_kernel(adj_ref, a_ref, b1_ref, w2_ref, g_ref):
    h = jnp.dot(adj_ref[...], a_ref[...], preferred_element_type=jnp.float32)
    h = jnp.maximum(h + b1_ref[...], 0.0)
    g_ref[...] = jnp.dot(h, w2_ref[...], preferred_element_type=jnp.float32)


def _pass2_kernel(adj_ref, g_ref, b2_ref, o_ref):
    z = jnp.dot(adj_ref[...], g_ref[...], preferred_element_type=jnp.float32)
    z = jnp.maximum(z + b2_ref[...], 0.0)
    m = jnp.max(z, axis=1, keepdims=True)
    s = z - m
    lse = jnp.log(jnp.sum(jnp.exp(s), axis=1, keepdims=True))
    o_ref[...] = s - lse


def _gcn_local(x, adj, w1, b1r, w2, b2r, axis_name=None):
    n_loc, n = adj.shape
    d_in = x.shape[1]
    hid = w1.shape[1]
    classes = w2.shape[1]

    a = pl.pallas_call(
        _xw_kernel,
        out_shape=jax.ShapeDtypeStruct((n, hid), jnp.float32),
    )(x, w1)

    br = 200
    grid = (n_loc // br,)

    g_loc = pl.pallas_call(
        _pass1_kernel,
        grid=grid,
        in_specs=[
            pl.BlockSpec((br, n), lambda i: (i, 0)),
            pl.BlockSpec((n, hid), lambda i: (0, 0)),
            pl.BlockSpec((1, hid), lambda i: (0, 0)),
            pl.BlockSpec((hid, classes), lambda i: (0, 0)),
        ],
        out_specs=pl.BlockSpec((br, classes), lambda i: (i, 0)),
        out_shape=jax.ShapeDtypeStruct((n_loc, classes), jnp.float32),
    )(adj, a, b1r, w2)

    if axis_name is not None:
        g = jax.lax.all_gather(g_loc, axis_name, axis=0, tiled=True)
    else:
        g = g_loc

    return pl.pallas_call(
        _pass2_kernel,
        grid=grid,
        in_specs=[
            pl.BlockSpec((br, n), lambda i: (i, 0)),
            pl.BlockSpec((n, classes), lambda i: (0, 0)),
            pl.BlockSpec((1, classes), lambda i: (0, 0)),
        ],
        out_specs=pl.BlockSpec((br, classes), lambda i: (i, 0)),
        out_shape=jax.ShapeDtypeStruct((n_loc, classes), jnp.float32),
    )(adj, g, b2r)


def kernel(x, adj, W1, b1, W2, b2):
    hid = W1.shape[1]
    classes = W2.shape[1]
    b1r = b1.reshape(1, hid)
    b2r = b2.reshape(1, classes)

    devs = [d for d in jax.devices() if d.platform == "tpu"]
    n = adj.shape[0]
    n_shards = 2 if (len(devs) >= 2 and n % 2 == 0) else 1

    if n_shards == 1:
        return _gcn_local(x, adj, W1, b1r, W2, b2r)

    mesh = Mesh(np.array(devs[:n_shards]), ("m",))
    fn = shard_map(
        functools.partial(_gcn_local, axis_name="m"),
        mesh=mesh,
        in_specs=(P(None, None), P("m", None), P(None, None),
                  P(None, None), P(None, None), P(None, None)),
        out_specs=P("m", None),
        check_rep=False,
    )
    return fn(x, adj, W1, b1r, W2, b2r)


# speedup vs baseline: 3.2319x; 3.2319x over previous
import jax
import jax.numpy as jnp
from jax.experimental import pallas as pl
from jax.experimental.pallas import tpu as pltpu

_BR = 200


def _gcn_kernel(adj_ref, x_ref, w1_ref, b1_ref, w2_ref, b2_ref, o_ref,
                a_scr, g_scr):
    p = pl.program_id(0)
    i = pl.program_id(1)

    @pl.when(jnp.logical_and(p == 0, i == 0))
    def _():
        a_scr[...] = jnp.dot(x_ref[...], w1_ref[...],
                             preferred_element_type=jnp.float32)

    @pl.when(p == 0)
    def _():
        h = jnp.dot(adj_ref[...], a_scr[...],
                    preferred_element_type=jnp.float32)
        h = jnp.maximum(h + b1_ref[...], 0.0)
        g_scr[pl.ds(i * _BR, _BR), :] = jnp.dot(
            h, w2_ref[...], preferred_element_type=jnp.float32)

    @pl.when(p == 1)
    def _():
        z = jnp.dot(adj_ref[...], g_scr[...],
                    preferred_element_type=jnp.float32)
        z = jnp.maximum(z + b2_ref[...], 0.0)
        m = jnp.max(z, axis=1, keepdims=True)
        s = z - m
        lse = jnp.log(jnp.sum(jnp.exp(s), axis=1, keepdims=True))
        o_ref[...] = s - lse


def kernel(x, adj, W1, b1, W2, b2):
    n, d_in = x.shape
    hid = W1.shape[1]
    classes = W2.shape[1]
    b1r = b1.reshape(1, hid)
    b2r = b2.reshape(1, classes)

    nb = n // _BR
    return pl.pallas_call(
        _gcn_kernel,
        grid=(2, nb),
        in_specs=[
            pl.BlockSpec((_BR, n), lambda p, i: (i, 0)),
            pl.BlockSpec((n, d_in), lambda p, i: (0, 0)),
            pl.BlockSpec((d_in, hid), lambda p, i: (0, 0)),
            pl.BlockSpec((1, hid), lambda p, i: (0, 0)),
            pl.BlockSpec((hid, classes), lambda p, i: (0, 0)),
            pl.BlockSpec((1, classes), lambda p, i: (0, 0)),
        ],
        out_specs=pl.BlockSpec((_BR, classes), lambda p, i: (i, 0)),
        out_shape=jax.ShapeDtypeStruct((n, classes), jnp.float32),
        scratch_shapes=[
            pltpu.VMEM((n, hid), jnp.float32),
            pltpu.VMEM((n, classes), jnp.float32),
        ],
    )(adj, x, W1, b1r, W2, b2r)


# speedup vs baseline: 3.3233x; 1.0283x over previous
import jax
import jax.numpy as jnp
from jax.experimental import pallas as pl
from jax.experimental.pallas import tpu as pltpu

_BR = 400


def _gcn_kernel(adj_ref, x_ref, w1_ref, b1_ref, w2_ref, b2_ref, o_ref,
                a_scr, g_scr):
    p = pl.program_id(0)
    i = pl.program_id(1)

    @pl.when(jnp.logical_and(p == 0, i == 0))
    def _():
        a_scr[...] = jnp.dot(x_ref[...], w1_ref[...],
                             preferred_element_type=jnp.float32)

    @pl.when(p == 0)
    def _():
        h = jnp.dot(adj_ref[...], a_scr[...],
                    preferred_element_type=jnp.float32)
        h = jnp.maximum(h + b1_ref[...], 0.0)
        g_scr[pl.ds(i * _BR, _BR), :] = jnp.dot(
            h, w2_ref[...], preferred_element_type=jnp.float32)

    @pl.when(p == 1)
    def _():
        z = jnp.dot(adj_ref[...], g_scr[...],
                    preferred_element_type=jnp.float32)
        z = jnp.maximum(z + b2_ref[...], 0.0)
        m = jnp.max(z, axis=1, keepdims=True)
        s = z - m
        lse = jnp.log(jnp.sum(jnp.exp(s), axis=1, keepdims=True))
        o_ref[...] = s - lse


def kernel(x, adj, W1, b1, W2, b2):
    n, d_in = x.shape
    hid = W1.shape[1]
    classes = W2.shape[1]
    b1r = b1.reshape(1, hid)
    b2r = b2.reshape(1, classes)

    nb = n // _BR
    return pl.pallas_call(
        _gcn_kernel,
        grid=(2, nb),
        in_specs=[
            pl.BlockSpec((_BR, n), lambda p, i: (i, 0)),
            pl.BlockSpec((n, d_in), lambda p, i: (0, 0)),
            pl.BlockSpec((d_in, hid), lambda p, i: (0, 0)),
            pl.BlockSpec((1, hid), lambda p, i: (0, 0)),
            pl.BlockSpec((hid, classes), lambda p, i: (0, 0)),
            pl.BlockSpec((1, classes), lambda p, i: (0, 0)),
        ],
        out_specs=pl.BlockSpec((_BR, classes), lambda p, i: (i, 0)),
        out_shape=jax.ShapeDtypeStruct((n, classes), jnp.float32),
        scratch_shapes=[
            pltpu.VMEM((n, hid), jnp.float32),
            pltpu.VMEM((n, classes), jnp.float32),
        ],
    )(adj, x, W1, b1r, W2, b2r)
